# baseline (speedup 1.0000x reference)
import functools

import jax
import jax.numpy as jnp
from jax import lax
from jax.experimental import pallas as pl
from jax.experimental.pallas import tpu as pltpu
from jax.experimental.pallas import tpu_sc as plsc

B = 16384
D = 64
NC = 2
NS = 16
NW = NC * NS
BPW = B // NW
CHUNK = 128
NCHUNK = BPW // CHUNK
VR = 128 // 16


def _mf_body(item_idx_hbm, user_idx_hbm, item_tab, user_tab, out_hbm,
             ii_v, ui_v, gi_v, gu_v, ibuf, ubuf, out_v,
             sem_i, sem_u):
    wid = lax.axis_index("s") * NC + lax.axis_index("c")
    base = wid * BPW

    pltpu.sync_copy(item_idx_hbm.at[pl.ds(base, BPW)], ii_v)
    pltpu.sync_copy(user_idx_hbm.at[pl.ds(base, BPW)], ui_v)

    iota16 = lax.iota(jnp.int32, 16)
    perm = {sh: iota16 ^ sh for sh in (8, 4, 2, 1)}

    for k in range(NCHUNK):
        cb = k * CHUNK
        for v in range(VR):
            gi_v[pl.ds(v * 16, 16)] = lax.shift_right_logical(
                ii_v[pl.ds(cb + v * 16, 16)], 1)
            gu_v[pl.ds(v * 16, 16)] = lax.shift_right_logical(
                ui_v[pl.ds(cb + v * 16, 16)], 1)
        cp_i = pltpu.async_copy(item_tab.at[gi_v], ibuf, sem_i)
        cp_u = pltpu.async_copy(user_tab.at[gu_v], ubuf, sem_u)
        cp_i.wait()
        cp_u.wait()

        def group_body(g, carry, k=k):
            rb = g * 16
            acc = jnp.zeros((16,), jnp.float32)
            oi_vec = (ii_v[pl.ds(cb + rb, 16)] & 1) * 64
            ou_vec = (ui_v[pl.ds(cb + rb, 16)] & 1) * 64
            for j in range(16):
                r = rb + j
                oi = oi_vec[j]
                ou = ou_vec[j]
                p = ibuf[r, pl.ds(oi, 16)] * ubuf[r, pl.ds(ou, 16)]
                p = p + ibuf[r, pl.ds(oi + 16, 16)] * ubuf[r, pl.ds(ou + 16, 16)]
                p = p + ibuf[r, pl.ds(oi + 32, 16)] * ubuf[r, pl.ds(ou + 32, 16)]
                p = p + ibuf[r, pl.ds(oi + 48, 16)] * ubuf[r, pl.ds(ou + 48, 16)]
                for sh in (8, 4, 2, 1):
                    p = p + p.at[perm[sh]].get(mode="promise_in_bounds")
                acc = jnp.where(iota16 == j, p, acc)
            out_v[pl.ds(cb + rb, 16)] = acc
            return carry

        lax.fori_loop(0, CHUNK // 16, group_body, 0)

    pltpu.sync_copy(out_v, out_hbm.at[pl.ds(base, BPW)])


@jax.jit
def _mf(item_idx, user_idx, item_table2, user_table2):
    mesh = plsc.VectorSubcoreMesh(core_axis_name="c", subcore_axis_name="s")
    kern = functools.partial(
        pl.kernel,
        mesh=mesh,
        compiler_params=pltpu.CompilerParams(use_tc_tiling_on_sc=True),
        out_type=jax.ShapeDtypeStruct((B,), jnp.float32),
        scratch_types=[
            pltpu.VMEM((BPW,), jnp.int32),
            pltpu.VMEM((BPW,), jnp.int32),
            pltpu.VMEM((CHUNK,), jnp.int32),
            pltpu.VMEM((CHUNK,), jnp.int32),
            pltpu.VMEM((CHUNK, 128), jnp.float32),
            pltpu.VMEM((CHUNK, 128), jnp.float32),
            pltpu.VMEM((BPW,), jnp.float32),
            pltpu.SemaphoreType.DMA,
            pltpu.SemaphoreType.DMA,
        ],
    )(_mf_body)
    return kern(item_idx, user_idx, item_table2, user_table2)


def kernel(item_vec, user_vec, item_table, user_table):
    item_table2 = item_table.reshape(-1, 128)
    user_table2 = user_table.reshape(-1, 128)
    return _mf(item_vec, user_vec, item_table2, user_table2)


# speedup vs baseline: 2.2637x; 2.2637x over previous
import functools

import jax
import jax.numpy as jnp
from jax import lax
from jax.experimental import pallas as pl
from jax.experimental.pallas import tpu as pltpu
from jax.experimental.pallas import tpu_sc as plsc

B = 16384
D = 64
NC = 2
NS = 16
NW = NC * NS
BPW = B // NW
G = 2
NG = BPW // G


def _mf_body(item_idx_hbm, user_idx_hbm, item_tab, user_tab, out_hbm,
             ii_v, ui_v, ibuf0, ubuf0, ibuf1, ubuf1, out_v, sem0, sem1):
    wid = lax.axis_index("s") * NC + lax.axis_index("c")
    base = wid * BPW

    pltpu.sync_copy(item_idx_hbm.at[pl.ds(base, BPW)], ii_v.at[pl.ds(0, BPW)])
    pltpu.sync_copy(user_idx_hbm.at[pl.ds(base, BPW)], ui_v.at[pl.ds(0, BPW)])

    iota16 = lax.iota(jnp.int32, 16)

    def fire(g, ibuf, ubuf, sem):
        iiv = ii_v[pl.ds(g * G, 16)]
        uiv = ui_v[pl.ds(g * G, 16)]
        for j in range(G):
            ci = pl.multiple_of((iiv[j] >> 7) * 128, 128)
            cu = pl.multiple_of((uiv[j] >> 7) * 128, 128)
            pltpu.async_copy(item_tab.at[:, pl.ds(ci, 128)], ibuf.at[j], sem)
            pltpu.async_copy(user_tab.at[:, pl.ds(cu, 128)], ubuf.at[j], sem)

    def drain_compute(g, ibuf, ubuf, sem, acc16):
        for _ in range(G):
            pltpu.make_async_copy(
                item_tab.at[:, pl.ds(0, 128)], ibuf.at[0], sem).wait()
            pltpu.make_async_copy(
                user_tab.at[:, pl.ds(0, 128)], ubuf.at[0], sem).wait()
        iiv = ii_v[pl.ds(g * G, 16)]
        uiv = ui_v[pl.ds(g * G, 16)]
        for j in range(G):
            li = iiv[j] & 127
            lu = uiv[j] & 127
            bi = (li >> 4) * 16
            bu = (lu >> 4) * 16
            pi = jnp.full((16,), li & 15, jnp.int32)
            pu = jnp.full((16,), lu & 15, jnp.int32)
            acc = jnp.zeros((16,), jnp.float32)
            for d in range(D):
                vi = ibuf[j, d, pl.ds(bi, 16)]
                vu = ubuf[j, d, pl.ds(bu, 16)]
                gi = vi.at[pi].get(mode="promise_in_bounds")
                gu = vu.at[pu].get(mode="promise_in_bounds")
                acc = acc + gi * gu
            lane = jnp.full((16,), (g * G + j) & 15, jnp.int32)
            acc16 = jnp.where(iota16 == lane, acc, acc16)
        return acc16

    fire(0, ibuf0, ubuf0, sem0)

    def pair_body(pp, acc16):
        g0 = pp * 2
        fire(g0 + 1, ibuf1, ubuf1, sem1)
        acc16 = drain_compute(g0, ibuf0, ubuf0, sem0, acc16)

        @pl.when(g0 + 2 < NG)
        def _():
            fire(g0 + 2, ibuf0, ubuf0, sem0)

        acc16 = drain_compute(g0 + 1, ibuf1, ubuf1, sem1, acc16)

        @pl.when((pp & 3) == 3)
        def _():
            out_v[pl.ds((pp >> 2) * 16, 16)] = acc16

        return acc16

    lax.fori_loop(0, NG // 2, pair_body, jnp.zeros((16,), jnp.float32))

    pltpu.sync_copy(out_v, out_hbm.at[pl.ds(base, BPW)])


@jax.jit
def _mf(item_idx, user_idx, item_tab_t, user_tab_t):
    mesh = plsc.VectorSubcoreMesh(core_axis_name="c", subcore_axis_name="s")
    kern = functools.partial(
        pl.kernel,
        mesh=mesh,
        compiler_params=pltpu.CompilerParams(use_tc_tiling_on_sc=True),
        out_type=jax.ShapeDtypeStruct((B,), jnp.float32),
        scratch_types=[
            pltpu.VMEM((BPW + 16, ), jnp.int32),
            pltpu.VMEM((BPW + 16, ), jnp.int32),
            pltpu.VMEM((G, D, 128), jnp.float32),
            pltpu.VMEM((G, D, 128), jnp.float32),
            pltpu.VMEM((G, D, 128), jnp.float32),
            pltpu.VMEM((G, D, 128), jnp.float32),
            pltpu.VMEM((BPW,), jnp.float32),
            pltpu.SemaphoreType.DMA,
            pltpu.SemaphoreType.DMA,
        ],
    )(_mf_body)
    return kern(item_idx, user_idx, item_tab_t, user_tab_t)


def kernel(item_vec, user_vec, item_table, user_table):
    return _mf(item_vec, user_vec, item_table.T, user_table.T)


# speedup vs baseline: 2.5029x; 1.1057x over previous
import functools

import jax
import jax.numpy as jnp
from jax import lax
from jax.experimental import pallas as pl
from jax.experimental.pallas import tpu as pltpu
from jax.experimental.pallas import tpu_sc as plsc

B = 16384
D = 64
NC = 2
NS = 16
NW = NC * NS
BPW = B // NW
G = 2
NG = BPW // G


def _mf_body(item_idx_hbm, user_idx_hbm, item_tab, user_tab, out_hbm,
             ii_v, ui_v, ibuf0, ubuf0, ibuf1, ubuf1, ibuf2, ubuf2,
             ibuf3, ubuf3, out_v, sem0, sem1, sem2, sem3):
    wid = lax.axis_index("s") * NC + lax.axis_index("c")
    base = wid * BPW

    pltpu.sync_copy(item_idx_hbm.at[pl.ds(base, BPW)], ii_v.at[pl.ds(0, BPW)])
    pltpu.sync_copy(user_idx_hbm.at[pl.ds(base, BPW)], ui_v.at[pl.ds(0, BPW)])

    iota16 = lax.iota(jnp.int32, 16)

    ibufs = (ibuf0, ibuf1, ibuf2, ibuf3)
    ubufs = (ubuf0, ubuf1, ubuf2, ubuf3)
    sems = (sem0, sem1, sem2, sem3)

    def fire(r, s):
        iiv = ii_v[pl.ds(r, 16)]
        uiv = ui_v[pl.ds(r, 16)]
        ci = pl.multiple_of((iiv[0] >> 7) * 128, 128)
        cu = pl.multiple_of((uiv[0] >> 7) * 128, 128)
        pltpu.async_copy(item_tab.at[:, pl.ds(ci, 128)], ibufs[s], sems[s])
        pltpu.async_copy(user_tab.at[:, pl.ds(cu, 128)], ubufs[s], sems[s])

    def drain_compute(r, s, acc16):
        pltpu.make_async_copy(
            item_tab.at[:, pl.ds(0, 128)], ibufs[s], sems[s]).wait()
        pltpu.make_async_copy(
            user_tab.at[:, pl.ds(0, 128)], ubufs[s], sems[s]).wait()
        ibuf, ubuf = ibufs[s], ubufs[s]
        iiv = ii_v[pl.ds(r, 16)]
        uiv = ui_v[pl.ds(r, 16)]
        li = iiv[0] & 127
        lu = uiv[0] & 127
        bi = (li >> 4) * 16
        bu = (lu >> 4) * 16
        pi = jnp.full((16,), li & 15, jnp.int32)
        pu = jnp.full((16,), lu & 15, jnp.int32)
        acc = jnp.zeros((16,), jnp.float32)
        for d in range(D):
            vi = ibuf[d, pl.ds(bi, 16)]
            vu = ubuf[d, pl.ds(bu, 16)]
            gi = vi.at[pi].get(mode="promise_in_bounds")
            gu = vu.at[pu].get(mode="promise_in_bounds")
            acc = acc + gi * gu
        lane = jnp.full((16,), r & 15, jnp.int32)
        return jnp.where(iota16 == lane, acc, acc16)

    for s in range(4):
        fire(s, s)

    def quad_body(q, acc16):
        r0 = q * 4
        for b in range(4):
            r = r0 + b
            acc16 = drain_compute(r, b, acc16)

            @pl.when(r + 4 < BPW)
            def _():
                fire(r + 4, b)

        @pl.when((q & 3) == 3)
        def _():
            out_v[pl.ds((q >> 2) * 16, 16)] = acc16

        return acc16

    lax.fori_loop(0, BPW // 4, quad_body, jnp.zeros((16,), jnp.float32))

    pltpu.sync_copy(out_v, out_hbm.at[pl.ds(base, BPW)])


@jax.jit
def _mf(item_idx, user_idx, item_tab_t, user_tab_t):
    mesh = plsc.VectorSubcoreMesh(core_axis_name="c", subcore_axis_name="s")
    kern = functools.partial(
        pl.kernel,
        mesh=mesh,
        compiler_params=pltpu.CompilerParams(use_tc_tiling_on_sc=True),
        out_type=jax.ShapeDtypeStruct((B,), jnp.float32),
        scratch_types=[
            pltpu.VMEM((BPW + 16, ), jnp.int32),
            pltpu.VMEM((BPW + 16, ), jnp.int32),
            pltpu.VMEM((D, 128), jnp.float32),
            pltpu.VMEM((D, 128), jnp.float32),
            pltpu.VMEM((D, 128), jnp.float32),
            pltpu.VMEM((D, 128), jnp.float32),
            pltpu.VMEM((D, 128), jnp.float32),
            pltpu.VMEM((D, 128), jnp.float32),
            pltpu.VMEM((D, 128), jnp.float32),
            pltpu.VMEM((D, 128), jnp.float32),
            pltpu.VMEM((BPW,), jnp.float32),
            pltpu.SemaphoreType.DMA,
            pltpu.SemaphoreType.DMA,
            pltpu.SemaphoreType.DMA,
            pltpu.SemaphoreType.DMA,
        ],
    )(_mf_body)
    return kern(item_idx, user_idx, item_tab_t, user_tab_t)


def kernel(item_vec, user_vec, item_table, user_table):
    return _mf(item_vec, user_vec, item_table.T, user_table.T)


# speedup vs baseline: 2.9332x; 1.1719x over previous
import functools

import jax
import jax.numpy as jnp
from jax import lax
from jax.experimental import pallas as pl
from jax.experimental.pallas import tpu as pltpu
from jax.experimental.pallas import tpu_sc as plsc

B = 16384
D = 64
NC = 2
NS = 16
NW = NC * NS
BPW = B // NW
G = 2
NG = BPW // G


def _mf_body(item_idx_hbm, user_idx_hbm, item_tab, user_tab, out_hbm,
             ii_v, ui_v, ibuf0, ubuf0, ibuf1, ubuf1, ibuf2, ubuf2,
             ibuf3, ubuf3, ibuf4, ubuf4, ibuf5, ubuf5, out_v,
             sem0, sem1, sem2, sem3, sem4, sem5):
    wid = lax.axis_index("s") * NC + lax.axis_index("c")
    base = wid * BPW

    pltpu.sync_copy(item_idx_hbm.at[pl.ds(base, BPW)], ii_v.at[pl.ds(0, BPW)])
    pltpu.sync_copy(user_idx_hbm.at[pl.ds(base, BPW)], ui_v.at[pl.ds(0, BPW)])

    iota16 = lax.iota(jnp.int32, 16)

    ibufs = (ibuf0, ibuf1, ibuf2, ibuf3, ibuf4, ibuf5)
    ubufs = (ubuf0, ubuf1, ubuf2, ubuf3, ubuf4, ubuf5)
    sems = (sem0, sem1, sem2, sem3, sem4, sem5)

    def fire(r, s):
        iiv = ii_v[pl.ds(r, 16)]
        uiv = ui_v[pl.ds(r, 16)]
        ci = pl.multiple_of((iiv[0] >> 7) * 128, 128)
        cu = pl.multiple_of((uiv[0] >> 7) * 128, 128)
        pltpu.async_copy(item_tab.at[:, pl.ds(ci, 128)], ibufs[s], sems[s])
        pltpu.async_copy(user_tab.at[:, pl.ds(cu, 128)], ubufs[s], sems[s])

    def drain_compute(r, s, acc16):
        pltpu.make_async_copy(
            item_tab.at[:, pl.ds(0, 128)], ibufs[s], sems[s]).wait()
        pltpu.make_async_copy(
            user_tab.at[:, pl.ds(0, 128)], ubufs[s], sems[s]).wait()
        ibuf, ubuf = ibufs[s], ubufs[s]
        iiv = ii_v[pl.ds(r, 16)]
        uiv = ui_v[pl.ds(r, 16)]
        li = iiv[0] & 127
        lu = uiv[0] & 127
        bi = (li >> 4) * 16
        bu = (lu >> 4) * 16
        pi = jnp.full((16,), li & 15, jnp.int32)
        pu = jnp.full((16,), lu & 15, jnp.int32)
        acc = jnp.zeros((16,), jnp.float32)
        for d in range(D):
            vi = ibuf[d, pl.ds(bi, 16)]
            vu = ubuf[d, pl.ds(bu, 16)]
            gi = vi.at[pi].get(mode="promise_in_bounds")
            gu = vu.at[pu].get(mode="promise_in_bounds")
            acc = acc + gi * gu
        lane = jnp.full((16,), r & 15, jnp.int32)
        acc16 = jnp.where(iota16 == lane, acc, acc16)

        @pl.when((r & 15) == 15)
        def _():
            out_v[pl.ds(r - 15, 16)] = acc16

        return acc16

    NSLOT = 6
    for s in range(NSLOT):
        fire(s, s)

    def ring_body(q, acc16):
        r0 = q * NSLOT
        for b in range(NSLOT):
            r = r0 + b
            acc16 = drain_compute(r, b, acc16)

            @pl.when(r + NSLOT < BPW)
            def _():
                fire(r + NSLOT, b)

        return acc16

    acc16 = lax.fori_loop(0, BPW // NSLOT, ring_body,
                          jnp.zeros((16,), jnp.float32))
    for r in range(BPW - BPW % NSLOT, BPW):
        acc16 = drain_compute(r, r % NSLOT, acc16)

    pltpu.sync_copy(out_v, out_hbm.at[pl.ds(base, BPW)])


@jax.jit
def _mf(item_idx, user_idx, item_tab_t, user_tab_t):
    mesh = plsc.VectorSubcoreMesh(core_axis_name="c", subcore_axis_name="s")
    kern = functools.partial(
        pl.kernel,
        mesh=mesh,
        compiler_params=pltpu.CompilerParams(use_tc_tiling_on_sc=True),
        out_type=jax.ShapeDtypeStruct((B,), jnp.float32),
        scratch_types=[
            pltpu.VMEM((BPW + 16, ), jnp.int32),
            pltpu.VMEM((BPW + 16, ), jnp.int32),
            pltpu.VMEM((D, 128), jnp.float32),
            pltpu.VMEM((D, 128), jnp.float32),
            pltpu.VMEM((D, 128), jnp.float32),
            pltpu.VMEM((D, 128), jnp.float32),
            pltpu.VMEM((D, 128), jnp.float32),
            pltpu.VMEM((D, 128), jnp.float32),
            pltpu.VMEM((D, 128), jnp.float32),
            pltpu.VMEM((D, 128), jnp.float32),
            pltpu.VMEM((D, 128), jnp.float32),
            pltpu.VMEM((D, 128), jnp.float32),
            pltpu.VMEM((D, 128), jnp.float32),
            pltpu.VMEM((D, 128), jnp.float32),
            pltpu.VMEM((BPW,), jnp.float32),
            pltpu.SemaphoreType.DMA,
            pltpu.SemaphoreType.DMA,
            pltpu.SemaphoreType.DMA,
            pltpu.SemaphoreType.DMA,
            pltpu.SemaphoreType.DMA,
            pltpu.SemaphoreType.DMA,
        ],
    )(_mf_body)
    return kern(item_idx, user_idx, item_tab_t, user_tab_t)


def kernel(item_vec, user_vec, item_table, user_table):
    return _mf(item_vec, user_vec, item_table.T, user_table.T)


# speedup vs baseline: 2.9625x; 1.0100x over previous
import functools

import jax
import jax.numpy as jnp
from jax import lax
from jax.experimental import pallas as pl
from jax.experimental.pallas import tpu as pltpu
from jax.experimental.pallas import tpu_sc as plsc

B = 16384
D = 64
NC = 2
NS = 16
NW = NC * NS
BPW = B // NW
G = 2
NG = BPW // G


def _mf_body(item_idx_hbm, user_idx_hbm, item_tab, user_tab, out_hbm,
             ii_v, ui_v, ibuf0, ubuf0, ibuf1, ubuf1, ibuf2, ubuf2,
             ibuf3, ubuf3, ibuf4, ubuf4, ibuf5, ubuf5, ibuf6, ubuf6, out_v,
             sem0, sem1, sem2, sem3, sem4, sem5, sem6):
    wid = lax.axis_index("s") * NC + lax.axis_index("c")
    base = wid * BPW

    pltpu.sync_copy(item_idx_hbm.at[pl.ds(base, BPW)], ii_v.at[pl.ds(0, BPW)])
    pltpu.sync_copy(user_idx_hbm.at[pl.ds(base, BPW)], ui_v.at[pl.ds(0, BPW)])

    iota16 = lax.iota(jnp.int32, 16)

    ibufs = (ibuf0, ibuf1, ibuf2, ibuf3, ibuf4, ibuf5, ibuf6)
    ubufs = (ubuf0, ubuf1, ubuf2, ubuf3, ubuf4, ubuf5, ubuf6)
    sems = (sem0, sem1, sem2, sem3, sem4, sem5, sem6)

    def fire(r, s):
        iiv = ii_v[pl.ds(r, 16)]
        uiv = ui_v[pl.ds(r, 16)]
        ci = pl.multiple_of((iiv[0] >> 7) * 128, 128)
        cu = pl.multiple_of((uiv[0] >> 7) * 128, 128)
        pltpu.async_copy(item_tab.at[:, pl.ds(ci, 128)], ibufs[s], sems[s])
        pltpu.async_copy(user_tab.at[:, pl.ds(cu, 128)], ubufs[s], sems[s])

    def drain_compute(r, s, acc16):
        pltpu.make_async_copy(
            item_tab.at[:, pl.ds(0, 128)], ibufs[s], sems[s]).wait()
        pltpu.make_async_copy(
            user_tab.at[:, pl.ds(0, 128)], ubufs[s], sems[s]).wait()
        ibuf, ubuf = ibufs[s], ubufs[s]
        iiv = ii_v[pl.ds(r, 16)]
        uiv = ui_v[pl.ds(r, 16)]
        li = iiv[0] & 127
        lu = uiv[0] & 127
        bi = (li >> 4) * 16
        bu = (lu >> 4) * 16
        pi = jnp.full((16,), li & 15, jnp.int32)
        pu = jnp.full((16,), lu & 15, jnp.int32)
        acc = jnp.zeros((16,), jnp.float32)
        for d in range(D):
            vi = ibuf[d, pl.ds(bi, 16)]
            vu = ubuf[d, pl.ds(bu, 16)]
            gi = vi.at[pi].get(mode="promise_in_bounds")
            gu = vu.at[pu].get(mode="promise_in_bounds")
            acc = acc + gi * gu
        lane = jnp.full((16,), r & 15, jnp.int32)
        acc16 = jnp.where(iota16 == lane, acc, acc16)

        @pl.when((r & 15) == 15)
        def _():
            out_v[pl.ds(r - 15, 16)] = acc16

        return acc16

    NSLOT = 7
    for s in range(NSLOT):
        fire(s, s)

    def ring_body(q, acc16):
        r0 = q * NSLOT
        for b in range(NSLOT):
            r = r0 + b
            acc16 = drain_compute(r, b, acc16)

            @pl.when(r + NSLOT < BPW)
            def _():
                fire(r + NSLOT, b)

        return acc16

    acc16 = lax.fori_loop(0, BPW // NSLOT, ring_body,
                          jnp.zeros((16,), jnp.float32))
    for r in range(BPW - BPW % NSLOT, BPW):
        acc16 = drain_compute(r, r % NSLOT, acc16)

    pltpu.sync_copy(out_v, out_hbm.at[pl.ds(base, BPW)])


@jax.jit
def _mf(item_idx, user_idx, item_tab_t, user_tab_t):
    mesh = plsc.VectorSubcoreMesh(core_axis_name="c", subcore_axis_name="s")
    kern = functools.partial(
        pl.kernel,
        mesh=mesh,
        compiler_params=pltpu.CompilerParams(use_tc_tiling_on_sc=True),
        out_type=jax.ShapeDtypeStruct((B,), jnp.float32),
        scratch_types=[
            pltpu.VMEM((BPW + 16, ), jnp.int32),
            pltpu.VMEM((BPW + 16, ), jnp.int32),
            pltpu.VMEM((D, 128), jnp.float32),
            pltpu.VMEM((D, 128), jnp.float32),
            pltpu.VMEM((D, 128), jnp.float32),
            pltpu.VMEM((D, 128), jnp.float32),
            pltpu.VMEM((D, 128), jnp.float32),
            pltpu.VMEM((D, 128), jnp.float32),
            pltpu.VMEM((D, 128), jnp.float32),
            pltpu.VMEM((D, 128), jnp.float32),
            pltpu.VMEM((D, 128), jnp.float32),
            pltpu.VMEM((D, 128), jnp.float32),
            pltpu.VMEM((D, 128), jnp.float32),
            pltpu.VMEM((D, 128), jnp.float32),
            pltpu.VMEM((D, 128), jnp.float32),
            pltpu.VMEM((D, 128), jnp.float32),
            pltpu.VMEM((BPW,), jnp.float32),
            pltpu.SemaphoreType.DMA,
            pltpu.SemaphoreType.DMA,
            pltpu.SemaphoreType.DMA,
            pltpu.SemaphoreType.DMA,
            pltpu.SemaphoreType.DMA,
            pltpu.SemaphoreType.DMA,
            pltpu.SemaphoreType.DMA,
        ],
    )(_mf_body)
    return kern(item_idx, user_idx, item_tab_t, user_tab_t)


def kernel(item_vec, user_vec, item_table, user_table):
    return _mf(item_vec, user_vec, item_table.T, user_table.T)
